# Initial kernel scaffold; baseline (speedup 1.0000x reference)
import jax
import jax.numpy as jnp
from jax.experimental import pallas as pl


def kernel(user_id, user_features, table, W1, b1, W2, b2, Wc, bc):
    raise NotImplementedError("write your pallas kernel here")


# speedup vs baseline: 1.3135x; 1.3135x over previous
import functools

import jax
import jax.numpy as jnp
from jax import lax
from jax.experimental import pallas as pl
from jax.experimental.pallas import tpu as pltpu
from jax.experimental.pallas import tpu_sc as plsc

EMBED_DIM = 64
FEAT_DIM = 64
BATCH = 16384
H1 = 32
H2 = 16

_SC_INFO = plsc.get_sparse_core_info()
_NC = _SC_INFO.num_cores
_NS = _SC_INFO.num_subcores
_NW = _NC * _NS
_B_PER_W = BATCH // _NW

_sc_mesh = plsc.VectorSubcoreMesh(core_axis_name="c", subcore_axis_name="s")


@functools.partial(
    pl.kernel,
    mesh=_sc_mesh,
    out_type=jax.ShapeDtypeStruct((BATCH, EMBED_DIM), jnp.float32),
    scratch_types=[
        pltpu.VMEM((_B_PER_W,), jnp.int32),
        pltpu.VMEM((_B_PER_W, EMBED_DIM), jnp.float32),
        pltpu.SemaphoreType.DMA,
    ],
    compiler_params=pltpu.CompilerParams(use_tc_tiling_on_sc=False),
)
def _sc_gather(table_hbm, idx_hbm, out_hbm, idx_v, rows_v, sem):
    wid = lax.axis_index("s") * _NC + lax.axis_index("c")
    base = wid * _B_PER_W
    pltpu.sync_copy(idx_hbm.at[pl.ds(base, _B_PER_W)], idx_v)
    pltpu.async_copy(table_hbm.at[idx_v], rows_v, sem).wait()
    pltpu.sync_copy(rows_v, out_hbm.at[pl.ds(base, _B_PER_W)])


_BLK = 1024


def _mlp_body(feat_ref, emb_ref, w1_ref, b1_ref, w2_ref, b2_ref, wc_ref,
              bc_ref, out_ref):
    h = jnp.dot(feat_ref[...], w1_ref[...], preferred_element_type=jnp.float32)
    h = jnp.maximum(h + b1_ref[...], 0.0)
    h = jnp.dot(h, w2_ref[...], preferred_element_type=jnp.float32)
    h = jnp.maximum(h + b2_ref[...], 0.0)
    y = jnp.dot(emb_ref[...], wc_ref[:EMBED_DIM, :],
                preferred_element_type=jnp.float32)
    y = y + jnp.dot(h, wc_ref[EMBED_DIM:, :],
                    preferred_element_type=jnp.float32)
    out_ref[...] = jnp.maximum(y + bc_ref[...], 0.0)


def _mlp(user_features, emb, W1, b1, W2, b2, Wc, bc):
    grid = (BATCH // _BLK,)
    return pl.pallas_call(
        _mlp_body,
        grid=grid,
        in_specs=[
            pl.BlockSpec((_BLK, FEAT_DIM), lambda i: (i, 0)),
            pl.BlockSpec((_BLK, EMBED_DIM), lambda i: (i, 0)),
            pl.BlockSpec((FEAT_DIM, H1), lambda i: (0, 0)),
            pl.BlockSpec((1, H1), lambda i: (0, 0)),
            pl.BlockSpec((H1, H2), lambda i: (0, 0)),
            pl.BlockSpec((1, H2), lambda i: (0, 0)),
            pl.BlockSpec((EMBED_DIM + H2, EMBED_DIM), lambda i: (0, 0)),
            pl.BlockSpec((1, EMBED_DIM), lambda i: (0, 0)),
        ],
        out_specs=pl.BlockSpec((_BLK, EMBED_DIM), lambda i: (i, 0)),
        out_shape=jax.ShapeDtypeStruct((BATCH, EMBED_DIM), jnp.float32),
    )(user_features, emb, W1, b1, W2, b2, Wc, bc)


def kernel(user_id, user_features, table, W1, b1, W2, b2, Wc, bc):
    idx = user_id.astype(jnp.int32)
    emb = _sc_gather(table, idx)
    return _mlp(user_features, emb, W1, b1.reshape(1, H1), W2,
                b2.reshape(1, H2), Wc, bc.reshape(1, EMBED_DIM))


# speedup vs baseline: 2.0170x; 1.5356x over previous
import functools

import jax
import jax.numpy as jnp
from jax import lax
from jax.experimental import pallas as pl
from jax.experimental.pallas import tpu as pltpu
from jax.experimental.pallas import tpu_sc as plsc

EMBED_DIM = 64
FEAT_DIM = 64
BATCH = 16384
H1 = 32
H2 = 16
EPAD = 128

_SC_INFO = plsc.get_sparse_core_info()
_NC = _SC_INFO.num_cores
_NS = _SC_INFO.num_subcores
_NW = _NC * _NS
_B_PER_W = BATCH // _NW

_sc_mesh = plsc.VectorSubcoreMesh(core_axis_name="c", subcore_axis_name="s")


@functools.partial(
    pl.kernel,
    mesh=_sc_mesh,
    out_type=jax.ShapeDtypeStruct((BATCH, EPAD), jnp.float32),
    scratch_types=[
        pltpu.VMEM((_B_PER_W,), jnp.int32),
        pltpu.VMEM((_B_PER_W, EPAD), jnp.float32),
        pltpu.SemaphoreType.DMA,
    ],
)
def _sc_gather(table_hbm, idx_hbm, out_hbm, idx_v, rows_v, sem):
    wid = lax.axis_index("s") * _NC + lax.axis_index("c")
    base = wid * _B_PER_W
    pltpu.sync_copy(idx_hbm.at[pl.ds(base, _B_PER_W)], idx_v)
    pltpu.async_copy(table_hbm.at[idx_v], rows_v, sem).wait()
    pltpu.sync_copy(rows_v, out_hbm.at[pl.ds(base, _B_PER_W)])


_BLK = 2048


def _mlp_body(uft_ref, emb_ref, w1t_ref, b1_ref, w2t_ref, b2_ref,
              wctt_ref, wcbt_ref, bc_ref, out_ref):
    f32 = jnp.float32
    h = lax.dot_general(w1t_ref[...], uft_ref[...], (((1,), (0,)), ((), ())),
                        preferred_element_type=f32)
    h = jnp.maximum(h + b1_ref[...], 0.0)
    h = lax.dot_general(w2t_ref[...], h, (((1,), (0,)), ((), ())),
                        preferred_element_type=f32)
    h = jnp.maximum(h + b2_ref[...], 0.0)
    y = lax.dot_general(wctt_ref[...], emb_ref[...], (((1,), (1,)), ((), ())),
                        preferred_element_type=f32)
    y = y + lax.dot_general(wcbt_ref[...], h, (((1,), (0,)), ((), ())),
                            preferred_element_type=f32)
    out_ref[...] = jnp.maximum(y + bc_ref[...], 0.0)


def _mlp(uft, emb_p, W1T, b1c, W2T, b2c, WcTopTp, WcBotT, bcc):
    grid = (BATCH // _BLK,)
    return pl.pallas_call(
        _mlp_body,
        grid=grid,
        in_specs=[
            pl.BlockSpec((FEAT_DIM, _BLK), lambda i: (0, i)),
            pl.BlockSpec((_BLK, EPAD), lambda i: (i, 0)),
            pl.BlockSpec((H1, FEAT_DIM), lambda i: (0, 0)),
            pl.BlockSpec((H1, 1), lambda i: (0, 0)),
            pl.BlockSpec((H2, H1), lambda i: (0, 0)),
            pl.BlockSpec((H2, 1), lambda i: (0, 0)),
            pl.BlockSpec((EMBED_DIM, EPAD), lambda i: (0, 0)),
            pl.BlockSpec((EMBED_DIM, H2), lambda i: (0, 0)),
            pl.BlockSpec((EMBED_DIM, 1), lambda i: (0, 0)),
        ],
        out_specs=pl.BlockSpec((EMBED_DIM, _BLK), lambda i: (0, i)),
        out_shape=jax.ShapeDtypeStruct((EMBED_DIM, BATCH), jnp.float32),
    )(uft, emb_p, W1T, b1c, W2T, b2c, WcTopTp, WcBotT, bcc)


def kernel(user_id, user_features, table, W1, b1, W2, b2, Wc, bc):
    idx = user_id.astype(jnp.int32)
    table_p = jnp.pad(table, ((0, 0), (0, EPAD - EMBED_DIM)))
    emb_p = _sc_gather(table_p, idx)
    WcTopTp = jnp.pad(Wc[:EMBED_DIM].T, ((0, 0), (0, EPAD - EMBED_DIM)))
    WcBotT = Wc[EMBED_DIM:].T
    outT = _mlp(user_features.T, emb_p, W1.T, b1.reshape(H1, 1), W2.T,
                b2.reshape(H2, 1), WcTopTp, WcBotT, bc.reshape(EMBED_DIM, 1))
    return outT.T
